# baseline (speedup 1.0000x reference)
import functools
import jax
import jax.numpy as jnp
from jax import lax
from jax.experimental import pallas as pl
from jax.experimental.pallas import tpu as pltpu

NN = 10000
EE = 320000
H = 128
NLAYER = 6
PW = 16

TE = 3200
TN = 1000


def _silu(x):
    return x * (1.0 / (1.0 + jnp.exp(-x)))


def _init_body(z_ref, embed_ref, a_ref, b_ref, h_ref, ha_ref, hb_ref):
    z = z_ref[...]
    onehot = (z == lax.broadcasted_iota(jnp.int32, (TN, H), 1)).astype(jnp.float32)
    h = jnp.dot(onehot, embed_ref[...], preferred_element_type=jnp.float32)
    h_ref[...] = h
    ha_ref[...] = jnp.dot(h, a_ref[...], preferred_element_type=jnp.float32)
    hb_ref[...] = jnp.dot(h, b_ref[...], preferred_element_type=jnp.float32)


def _init_call(z2d, embed_pad, a0, b0):
    grid = NN // TN
    wspec = pl.BlockSpec((H, H), lambda i: (0, 0))
    return pl.pallas_call(
        _init_body,
        grid=(grid,),
        in_specs=[
            pl.BlockSpec((TN, 1), lambda i: (i, 0)),
            wspec, wspec, wspec,
        ],
        out_specs=[
            pl.BlockSpec((TN, H), lambda i: (i, 0)),
            pl.BlockSpec((TN, H), lambda i: (i, 0)),
            pl.BlockSpec((TN, H), lambda i: (i, 0)),
        ],
        out_shape=[
            jax.ShapeDtypeStruct((NN, H), jnp.float32),
            jax.ShapeDtypeStruct((NN, H), jnp.float32),
            jax.ShapeDtypeStruct((NN, H), jnp.float32),
        ],
    )(z2d, embed_pad, a0, b0)


def _edge_body(ga_ref, gb_ref, pr_ref, pc_ref, wd_ref, eb1_ref, ew2_ref,
               eb2_ref, cw1_ref, cb1_ref, cw2_ref, m_ref, cwcd_ref):
    diff = pr_ref[...] - pc_ref[...]
    d2 = jnp.sum(diff * diff, axis=1, keepdims=True)
    dist = jnp.sqrt(d2 + 1e-8)
    pre1 = ga_ref[...] + gb_ref[...] + dist * wd_ref[...] + eb1_ref[...]
    t1 = _silu(pre1)
    m = _silu(jnp.dot(t1, ew2_ref[...], preferred_element_type=jnp.float32)
              + eb2_ref[...])
    u = _silu(jnp.dot(m, cw1_ref[...], preferred_element_type=jnp.float32)
              + cb1_ref[...])
    cw = jnp.dot(u, cw2_ref[...], preferred_element_type=jnp.float32)
    cd = diff / (dist + 1e-8)
    m_ref[...] = m
    cwcd_ref[...] = cw * cd


def _edge_call(ga, gb, pr, pc, wd, eb1, ew2, eb2, cw1, cb1, cw2):
    grid = EE // TE
    wspec = pl.BlockSpec((H, H), lambda i: (0, 0))
    rspec = pl.BlockSpec((1, H), lambda i: (0, 0))
    return pl.pallas_call(
        _edge_body,
        grid=(grid,),
        in_specs=[
            pl.BlockSpec((TE, H), lambda i: (i, 0)),
            pl.BlockSpec((TE, H), lambda i: (i, 0)),
            pl.BlockSpec((TE, PW), lambda i: (i, 0)),
            pl.BlockSpec((TE, PW), lambda i: (i, 0)),
            rspec, rspec, wspec, rspec, wspec, rspec,
            pl.BlockSpec((H, 1), lambda i: (0, 0)),
        ],
        out_specs=[
            pl.BlockSpec((TE, H), lambda i: (i, 0)),
            pl.BlockSpec((TE, PW), lambda i: (i, 0)),
        ],
        out_shape=[
            jax.ShapeDtypeStruct((EE, H), jnp.float32),
            jax.ShapeDtypeStruct((EE, PW), jnp.float32),
        ],
    )(ga, gb, pr, pc, wd, eb1, ew2, eb2, cw1, cb1, cw2)


def _node_body(h_ref, agg0_ref, agg1_ref, dp0_ref, dp1_ref, p_ref,
               nw1h_ref, nw1a_ref, nb1_ref, nw2_ref, nb2_ref, g_ref, b_ref,
               an_ref, bn_ref,
               h_out, p_out, ha_out, hb_out):
    h = h_ref[...]
    agg = agg0_ref[...] + agg1_ref[...]
    pre = (jnp.dot(h, nw1h_ref[...], preferred_element_type=jnp.float32)
           + jnp.dot(agg, nw1a_ref[...], preferred_element_type=jnp.float32)
           + nb1_ref[...])
    hn = jnp.dot(_silu(pre), nw2_ref[...], preferred_element_type=jnp.float32) \
        + nb2_ref[...]
    x = h + hn
    mu = jnp.mean(x, axis=1, keepdims=True)
    xc = x - mu
    var = jnp.mean(xc * xc, axis=1, keepdims=True)
    hnew = g_ref[...] * xc * lax.rsqrt(var + 1e-5) + b_ref[...]
    h_out[...] = hnew
    p_out[...] = p_ref[...] + dp0_ref[...] + dp1_ref[...]
    ha_out[...] = jnp.dot(hnew, an_ref[...], preferred_element_type=jnp.float32)
    hb_out[...] = jnp.dot(hnew, bn_ref[...], preferred_element_type=jnp.float32)


def _node_call(h, agg0, agg1, dp0, dp1, p, nw1h, nw1a, nb1, nw2, nb2, g, b,
               a_next, b_next):
    grid = NN // TN
    nspec = pl.BlockSpec((TN, H), lambda i: (i, 0))
    pspec = pl.BlockSpec((TN, PW), lambda i: (i, 0))
    wspec = pl.BlockSpec((H, H), lambda i: (0, 0))
    rspec = pl.BlockSpec((1, H), lambda i: (0, 0))
    return pl.pallas_call(
        _node_body,
        grid=(grid,),
        in_specs=[
            nspec, nspec, nspec, pspec, pspec, pspec,
            wspec, wspec, rspec, wspec, rspec, rspec, rspec,
            wspec, wspec,
        ],
        out_specs=[nspec, pspec, nspec, nspec],
        out_shape=[
            jax.ShapeDtypeStruct((NN, H), jnp.float32),
            jax.ShapeDtypeStruct((NN, PW), jnp.float32),
            jax.ShapeDtypeStruct((NN, H), jnp.float32),
            jax.ShapeDtypeStruct((NN, H), jnp.float32),
        ],
    )(h, agg0, agg1, dp0, dp1, p, nw1h, nw1a, nb1, nw2, nb2, g, b,
      a_next, b_next)


def _readout_body(h_ref, rw1_ref, rb1_ref, rw2_ref, rb2_ref, out_ref):
    t = _silu(jnp.dot(h_ref[...], rw1_ref[...],
                      preferred_element_type=jnp.float32) + rb1_ref[...])
    out_ref[...] = jnp.dot(t, rw2_ref[...],
                           preferred_element_type=jnp.float32) + rb2_ref[...]


def _readout_call(h, rw1, rb1, rw2, rb2):
    grid = NN // TN
    return pl.pallas_call(
        _readout_body,
        grid=(grid,),
        in_specs=[
            pl.BlockSpec((TN, H), lambda i: (i, 0)),
            pl.BlockSpec((H, H), lambda i: (0, 0)),
            pl.BlockSpec((1, H), lambda i: (0, 0)),
            pl.BlockSpec((H, 1), lambda i: (0, 0)),
            pl.BlockSpec((1, 1), lambda i: (0, 0)),
        ],
        out_specs=pl.BlockSpec((TN, 1), lambda i: (i, 0)),
        out_shape=jax.ShapeDtypeStruct((NN, 1), jnp.float32),
    )(h, rw1, rb1, rw2, rb2)


def _gather_edges(ha, hb, ptab, row, col):
    ga = jnp.take(ha, row, axis=0)
    gb = jnp.take(hb, col, axis=0)
    pr = jnp.take(ptab, row, axis=0)
    pc = jnp.take(ptab, col, axis=0)
    return ga, gb, pr, pc


def _scatter_edges(m, cwcd, row):
    agg = jnp.zeros((NN, H), jnp.float32).at[row].add(m)
    dp = jnp.zeros((NN, PW), jnp.float32).at[row].add(cwcd)
    z = jnp.zeros_like(agg)
    zp = jnp.zeros_like(dp)
    return agg, z, dp, zp


def kernel(z, pos, edge_index, embed, ew1, eb1, ew2, eb2, cw1, cb1, cw2,
           nw1, nb1, nw2, nb2, ln_g, ln_b, rw1, rb1, rw2, rb2):
    row = edge_index[0].astype(jnp.int32)
    col = edge_index[1].astype(jnp.int32)
    z2d = z.astype(jnp.int32).reshape(NN, 1)
    embed_pad = jnp.zeros((H, H), jnp.float32).at[:embed.shape[0]].set(embed)
    ptab = jnp.zeros((NN, PW), jnp.float32).at[:, :3].set(pos)

    ew1_a = ew1[:, :H, :]
    ew1_b = ew1[:, H:2 * H, :]
    ew1_d = ew1[:, 2 * H:, :]
    nw1_h = nw1[:, :H, :]
    nw1_a = nw1[:, H:, :]

    h, ha, hb = _init_call(z2d, embed_pad, ew1_a[0], ew1_b[0])

    for l in range(NLAYER):
        ga, gb, pr, pc = _gather_edges(ha, hb, ptab, row, col)
        m, cwcd = _edge_call(ga, gb, pr, pc, ew1_d[l], eb1[l].reshape(1, H),
                             ew2[l], eb2[l].reshape(1, H), cw1[l],
                             cb1[l].reshape(1, H), cw2[l])
        agg0, agg1, dp0, dp1 = _scatter_edges(m, cwcd, row)
        ln = (l + 1) % NLAYER
        h, ptab, ha, hb = _node_call(
            h, agg0, agg1, dp0, dp1, ptab,
            nw1_h[l], nw1_a[l], nb1[l].reshape(1, H), nw2[l],
            nb2[l].reshape(1, H), ln_g[l].reshape(1, H), ln_b[l].reshape(1, H),
            ew1_a[ln], ew1_b[ln])

    return _readout_call(h, rw1, rb1.reshape(1, H), rw2, rb2.reshape(1, 1))


# speedup vs baseline: 3.8940x; 3.8940x over previous
import functools
import jax
import jax.numpy as jnp
from jax import lax
from jax.experimental import pallas as pl
from jax.experimental.pallas import tpu as pltpu
from jax.experimental.pallas import tpu_sc as plsc

NN = 10000
EE = 320000
H = 128
NLAYER = 6
PW = 8

TE = 3200
TN = 1000


def _silu(x):
    return x * (1.0 / (1.0 + jnp.exp(-x)))


def _init_body(z_ref, embed_ref, a_ref, b_ref, h_ref, ha_ref, hb_ref):
    z = z_ref[...]
    onehot = (z == lax.broadcasted_iota(jnp.int32, (TN, H), 1)).astype(jnp.float32)
    h = jnp.dot(onehot, embed_ref[...], preferred_element_type=jnp.float32)
    h_ref[...] = h
    ha_ref[...] = jnp.dot(h, a_ref[...], preferred_element_type=jnp.float32)
    hb_ref[...] = jnp.dot(h, b_ref[...], preferred_element_type=jnp.float32)


def _init_call(z2d, embed_pad, a0, b0):
    grid = NN // TN
    wspec = pl.BlockSpec((H, H), lambda i: (0, 0))
    return pl.pallas_call(
        _init_body,
        grid=(grid,),
        in_specs=[
            pl.BlockSpec((TN, 1), lambda i: (i, 0)),
            wspec, wspec, wspec,
        ],
        out_specs=[
            pl.BlockSpec((TN, H), lambda i: (i, 0)),
            pl.BlockSpec((TN, H), lambda i: (i, 0)),
            pl.BlockSpec((TN, H), lambda i: (i, 0)),
        ],
        out_shape=[
            jax.ShapeDtypeStruct((NN, H), jnp.float32),
            jax.ShapeDtypeStruct((NN, H), jnp.float32),
            jax.ShapeDtypeStruct((NN, H), jnp.float32),
        ],
    )(z2d, embed_pad, a0, b0)


def _edge_body(ga_ref, gb_ref, diff_ref, wd_ref, eb1_ref, ew2_ref,
               eb2_ref, cw1_ref, cb1_ref, cw2_ref, m_ref, cwf_ref):
    diff = diff_ref[...]
    d2 = jnp.sum(diff * diff, axis=1, keepdims=True)
    dist = jnp.sqrt(d2 + 1e-8)
    pre1 = ga_ref[...] + gb_ref[...] + dist * wd_ref[...] + eb1_ref[...]
    t1 = _silu(pre1)
    m = _silu(jnp.dot(t1, ew2_ref[...], preferred_element_type=jnp.float32)
              + eb2_ref[...])
    u = _silu(jnp.dot(m, cw1_ref[...], preferred_element_type=jnp.float32)
              + cb1_ref[...])
    cw = jnp.dot(u, cw2_ref[...], preferred_element_type=jnp.float32)
    cd = diff / (dist + 1e-8)
    m_ref[...] = m
    cwf_ref[...] = jnp.concatenate(
        [cw * cd, jnp.zeros((TE, H - PW), jnp.float32)], axis=1)


def _edge_call(ga, gb, diff, wd, eb1, ew2, eb2, cw1, cb1, cw2):
    grid = EE // TE
    wspec = pl.BlockSpec((H, H), lambda i: (0, 0))
    rspec = pl.BlockSpec((1, H), lambda i: (0, 0))
    return pl.pallas_call(
        _edge_body,
        grid=(grid,),
        in_specs=[
            pl.BlockSpec((TE, H), lambda i: (i, 0)),
            pl.BlockSpec((TE, H), lambda i: (i, 0)),
            pl.BlockSpec((TE, PW), lambda i: (i, 0)),
            rspec, rspec, wspec, rspec, wspec, rspec,
            pl.BlockSpec((H, 1), lambda i: (0, 0)),
        ],
        out_specs=[
            pl.BlockSpec((TE, H), lambda i: (i, 0)),
            pl.BlockSpec((TE, H), lambda i: (i, 0)),
        ],
        out_shape=[
            jax.ShapeDtypeStruct((EE, H), jnp.float32),
            jax.ShapeDtypeStruct((EE, H), jnp.float32),
        ],
    )(ga, gb, diff, wd, eb1, ew2, eb2, cw1, cb1, cw2)


def _node_body(h_ref, agg0_ref, agg1_ref, dp0_ref, dp1_ref, p_ref,
               nw1h_ref, nw1a_ref, nb1_ref, nw2_ref, nb2_ref, g_ref, b_ref,
               an_ref, bn_ref,
               h_out, p_out, ha_out, hb_out):
    h = h_ref[...]
    agg = agg0_ref[...] + agg1_ref[...]
    pre = (jnp.dot(h, nw1h_ref[...], preferred_element_type=jnp.float32)
           + jnp.dot(agg, nw1a_ref[...], preferred_element_type=jnp.float32)
           + nb1_ref[...])
    hn = jnp.dot(_silu(pre), nw2_ref[...], preferred_element_type=jnp.float32) \
        + nb2_ref[...]
    x = h + hn
    mu = jnp.mean(x, axis=1, keepdims=True)
    xc = x - mu
    var = jnp.mean(xc * xc, axis=1, keepdims=True)
    hnew = g_ref[...] * xc * lax.rsqrt(var + 1e-5) + b_ref[...]
    h_out[...] = hnew
    dp = dp0_ref[...] + dp1_ref[...]
    p_out[...] = p_ref[...] + dp[:, :PW]
    ha_out[...] = jnp.dot(hnew, an_ref[...], preferred_element_type=jnp.float32)
    hb_out[...] = jnp.dot(hnew, bn_ref[...], preferred_element_type=jnp.float32)


def _node_call(h, agg0, agg1, dp0, dp1, p, nw1h, nw1a, nb1, nw2, nb2, g, b,
               a_next, b_next):
    grid = NN // TN
    nspec = pl.BlockSpec((TN, H), lambda i: (i, 0))
    pspec = pl.BlockSpec((TN, PW), lambda i: (i, 0))
    wspec = pl.BlockSpec((H, H), lambda i: (0, 0))
    rspec = pl.BlockSpec((1, H), lambda i: (0, 0))
    return pl.pallas_call(
        _node_body,
        grid=(grid,),
        in_specs=[
            nspec, nspec, nspec, nspec, nspec, pspec,
            wspec, wspec, rspec, wspec, rspec, rspec, rspec,
            wspec, wspec,
        ],
        out_specs=[nspec, pspec, nspec, nspec],
        out_shape=[
            jax.ShapeDtypeStruct((NN, H), jnp.float32),
            jax.ShapeDtypeStruct((NN, PW), jnp.float32),
            jax.ShapeDtypeStruct((NN, H), jnp.float32),
            jax.ShapeDtypeStruct((NN, H), jnp.float32),
        ],
    )(h, agg0, agg1, dp0, dp1, p, nw1h, nw1a, nb1, nw2, nb2, g, b,
      a_next, b_next)


def _readout_body(h_ref, rw1_ref, rb1_ref, rw2_ref, rb2_ref, out_ref):
    t = _silu(jnp.dot(h_ref[...], rw1_ref[...],
                      preferred_element_type=jnp.float32) + rb1_ref[...])
    out_ref[...] = jnp.dot(t, rw2_ref[...],
                           preferred_element_type=jnp.float32) + rb2_ref[...]


def _readout_call(h, rw1, rb1, rw2, rb2):
    grid = NN // TN
    return pl.pallas_call(
        _readout_body,
        grid=(grid,),
        in_specs=[
            pl.BlockSpec((TN, H), lambda i: (i, 0)),
            pl.BlockSpec((H, H), lambda i: (0, 0)),
            pl.BlockSpec((1, H), lambda i: (0, 0)),
            pl.BlockSpec((H, 1), lambda i: (0, 0)),
            pl.BlockSpec((1, 1), lambda i: (0, 0)),
        ],
        out_specs=pl.BlockSpec((TN, 1), lambda i: (i, 0)),
        out_shape=jax.ShapeDtypeStruct((NN, 1), jnp.float32),
    )(h, rw1, rb1, rw2, rb2)


NC = 2
NS = 16
NTILE = NC * NS
CH = 128
NCHUNK = EE // CH
SPLIT = 640
LAST = NN - (NS - 1) * SPLIT
LASTR = LAST - (LAST // CH) * CH

_sc_mesh = plsc.VectorSubcoreMesh(
    core_axis_name="c", subcore_axis_name="s", num_cores=NC, num_subcores=NS)

_IOTA16 = None


@functools.partial(
    pl.kernel,
    out_type=[
        jax.ShapeDtypeStruct((EE, H), jnp.float32),
        jax.ShapeDtypeStruct((EE, H), jnp.float32),
        jax.ShapeDtypeStruct((EE * PW,), jnp.float32),
    ],
    mesh=_sc_mesh,
    scratch_types=[
        pltpu.VMEM((NN * PW,), jnp.float32),
        pltpu.VMEM((CH,), jnp.int32),
        pltpu.VMEM((CH,), jnp.int32),
        pltpu.VMEM((CH, H), jnp.float32),
        pltpu.VMEM((CH, H), jnp.float32),
        pltpu.VMEM((CH * PW,), jnp.float32),
        pltpu.SemaphoreType.DMA,
    ],
    compiler_params=pltpu.CompilerParams(needs_layout_passes=False),
)
def _sc_gather(ha, hb, ptab, row, col, ga, gb, dout,
               ptv, idxr, idxc, gav, gbv, dv, sem):
    c = lax.axis_index("c")
    s = lax.axis_index("s")
    wid = s * NC + c
    nch = NCHUNK // NTILE + jnp.where(wid < NCHUNK % NTILE, 1, 0)

    pltpu.sync_copy(ptab, ptv)

    iota16 = lax.iota(jnp.int32, 16)
    def zdv(i, _):
        dv[pl.ds(i * 16, 16)] = jnp.zeros((16,), jnp.float32)
        return 0
    lax.fori_loop(0, CH * PW // 16, zdv, 0)

    def body(k, _):
        base = (wid + k * NTILE) * CH
        pltpu.sync_copy(row.at[pl.ds(base, CH)], idxr)
        pltpu.sync_copy(col.at[pl.ds(base, CH)], idxc)
        cp1 = pltpu.async_copy(ha.at[idxr], gav, sem)
        cp2 = pltpu.async_copy(hb.at[idxc], gbv, sem)
        for g in range(CH // 16):
            ir16 = idxr[pl.ds(g * 16, 16)]
            ic16 = idxc[pl.ds(g * 16, 16)]
            flat16 = (g * 16 + iota16) * PW
            for j in range(3):
                jj = jnp.full((16,), j, jnp.int32)
                a = plsc.load_gather(ptv, [ir16 * PW + jj])
                b = plsc.load_gather(ptv, [ic16 * PW + jj])
                plsc.store_scatter(dv, [flat16 + jj], a - b)
        pltpu.sync_copy(dv, dout.at[pl.ds(base * PW, CH * PW)])
        cp1.wait()
        cp2.wait()
        pltpu.sync_copy(gav, ga.at[pl.ds(base, CH)])
        pltpu.sync_copy(gbv, gb.at[pl.ds(base, CH)])
        return 0

    lax.fori_loop(0, nch, body, 0)


@functools.partial(
    pl.kernel,
    out_type=[
        jax.ShapeDtypeStruct((NC, NN, H), jnp.float32),
        jax.ShapeDtypeStruct((NC, NN, H), jnp.float32),
    ],
    mesh=_sc_mesh,
    scratch_types=[
        pltpu.VMEM((CH,), jnp.int32),
        pltpu.VMEM((CH, H), jnp.float32),
        pltpu.VMEM_SHARED((NN, H), jnp.float32),
    ],
)
def _sc_scatter(m, cwf, row, aggp, dpp,
                idxv, mv, aggsh):
    c = lax.axis_index("c")
    s = lax.axis_index("s")

    zeros16 = jnp.zeros((16,), jnp.float32)
    nfull = jnp.where(s < NS - 1, SPLIT // CH, LAST // CH)

    def zero_mv():
        def zrow(i, _):
            def zcol(j, _2):
                mv[i, pl.ds(j * 16, 16)] = zeros16
                return 0
            lax.fori_loop(0, H // 16, zcol, 0)
            return 0
        lax.fori_loop(0, CH, zrow, 0)

    def zero_shared():
        def zq(q, _):
            pltpu.sync_copy(mv, aggsh.at[pl.ds(s * SPLIT + q * CH, CH)])
            return 0
        lax.fori_loop(0, nfull, zq, 0)

        @pl.when(s == NS - 1)
        def _():
            pltpu.sync_copy(mv.at[pl.ds(0, LASTR)],
                            aggsh.at[pl.ds(s * SPLIT + LAST - LASTR, LASTR)])

    half = NCHUNK // NC
    nch = half // NS + jnp.where(s < half % NS, 1, 0)

    def scatter_phase(src):
        def body(k, _):
            base = (c * half + s + k * NS) * CH
            pltpu.sync_copy(row.at[pl.ds(base, CH)], idxv)
            pltpu.sync_copy(src.at[pl.ds(base, CH)], mv)
            pltpu.sync_copy(mv, aggsh.at[idxv], add=True)
            return 0
        lax.fori_loop(0, nch, body, 0)

    def writeback(dst):
        def wq(q, _):
            pltpu.sync_copy(aggsh.at[pl.ds(s * SPLIT + q * CH, CH)], mv)
            pltpu.sync_copy(mv, dst.at[c, pl.ds(s * SPLIT + q * CH, CH)])
            return 0
        lax.fori_loop(0, nfull, wq, 0)

        @pl.when(s == NS - 1)
        def _():
            off = s * SPLIT + LAST - LASTR
            pltpu.sync_copy(aggsh.at[pl.ds(off, LASTR)],
                            mv.at[pl.ds(0, LASTR)])
            pltpu.sync_copy(mv.at[pl.ds(0, LASTR)],
                            dst.at[c, pl.ds(off, LASTR)])

    zero_mv()
    zero_shared()
    plsc.subcore_barrier()
    scatter_phase(m)
    plsc.subcore_barrier()
    writeback(aggp)
    zero_mv()
    zero_shared()
    plsc.subcore_barrier()
    scatter_phase(cwf)
    plsc.subcore_barrier()
    writeback(dpp)


def _gather_edges(ha, hb, ptab, row, col):
    ga, gb, dflat = _sc_gather(ha, hb, ptab.reshape(NN * PW), row, col)
    return ga, gb, dflat.reshape(EE, PW)


def _scatter_edges(m, cwf, row):
    aggp, dpp = _sc_scatter(m, cwf, row)
    return aggp[0], aggp[1], dpp[0], dpp[1]


def kernel(z, pos, edge_index, embed, ew1, eb1, ew2, eb2, cw1, cb1, cw2,
           nw1, nb1, nw2, nb2, ln_g, ln_b, rw1, rb1, rw2, rb2):
    row = edge_index[0].astype(jnp.int32)
    col = edge_index[1].astype(jnp.int32)
    z2d = z.astype(jnp.int32).reshape(NN, 1)
    embed_pad = jnp.zeros((H, H), jnp.float32).at[:embed.shape[0]].set(embed)
    ptab = jnp.zeros((NN, PW), jnp.float32).at[:, :3].set(pos)

    ew1_a = ew1[:, :H, :]
    ew1_b = ew1[:, H:2 * H, :]
    ew1_d = ew1[:, 2 * H:, :]
    nw1_h = nw1[:, :H, :]
    nw1_a = nw1[:, H:, :]

    h, ha, hb = _init_call(z2d, embed_pad, ew1_a[0], ew1_b[0])

    for l in range(NLAYER):
        ga, gb, diff = _gather_edges(ha, hb, ptab, row, col)
        m, cwf = _edge_call(ga, gb, diff, ew1_d[l], eb1[l].reshape(1, H),
                            ew2[l], eb2[l].reshape(1, H), cw1[l],
                            cb1[l].reshape(1, H), cw2[l])
        agg0, agg1, dp0, dp1 = _scatter_edges(m, cwf, row)
        ln = (l + 1) % NLAYER
        h, ptab, ha, hb = _node_call(
            h, agg0, agg1, dp0, dp1, ptab,
            nw1_h[l], nw1_a[l], nb1[l].reshape(1, H), nw2[l],
            nb2[l].reshape(1, H), ln_g[l].reshape(1, H), ln_b[l].reshape(1, H),
            ew1_a[ln], ew1_b[ln])

    return _readout_call(h, rw1, rb1.reshape(1, H), rw2, rb2.reshape(1, 1))
